# baseline (speedup 1.0000x reference)
import math
import functools

import jax
import jax.numpy as jnp
from jax import lax
from jax.experimental import pallas as pl
from jax.experimental.pallas import tpu as pltpu
from jax.experimental.pallas import tpu_sc as plsc

VOCAB = 1000000
EMBED_DIM = 64
B, L = 4096, 200
SCALE = math.sqrt(EMBED_DIM)

NC, NS = 2, 16
NW = NC * NS

ROWS = B * L
IDXW = 100
CHUNK = 800
GATHERS = CHUNK // IDXW
ROWS_PER_W = ROWS // NW
CHUNKS_PER_W = ROWS_PER_W // CHUNK
NSUB = EMBED_DIM // 16


def _pe_table():
    idx = jnp.arange(0, EMBED_DIM, 2, dtype=jnp.float32)
    pos = jnp.arange(0, L, dtype=jnp.float32)[:, None]
    div_term = jnp.exp(-idx / EMBED_DIM * math.log(10000.0))
    ang = pos * div_term
    pe = jnp.zeros((L, EMBED_DIM), dtype=jnp.float32)
    pe = pe.at[:, 0::2].set(jnp.sin(ang))
    pe = pe.at[:, 1::2].set(jnp.cos(ang))
    return pe


@functools.partial(
    pl.kernel,
    out_type=jax.ShapeDtypeStruct((ROWS, EMBED_DIM), jnp.float32),
    mesh=plsc.VectorSubcoreMesh(core_axis_name="c", subcore_axis_name="s"),
    scratch_types=[
        pltpu.VMEM((GATHERS, IDXW), jnp.int32),
        pltpu.VMEM((CHUNK, EMBED_DIM), jnp.float32),
        pltpu.VMEM((L, EMBED_DIM), jnp.float32),
        pltpu.SemaphoreType.DMA,
    ],
    compiler_params=pltpu.CompilerParams(use_tc_tiling_on_sc=False),
)
def _sc_embed(x_hbm, w_hbm, pe_hbm, out_hbm, idx_v, rows_v, pe_v, sem):
    wid = lax.axis_index("s") * NC + lax.axis_index("c")
    base_idxrow = wid * (ROWS_PER_W // IDXW)
    base_row = wid * ROWS_PER_W

    pltpu.sync_copy(pe_hbm, pe_v)

    def chunk_body(c, carry):
        idxrow0 = base_idxrow + c * GATHERS
        row0 = base_row + c * CHUNK

        pltpu.sync_copy(x_hbm.at[pl.ds(idxrow0, GATHERS)], idx_v)

        copies = [
            pltpu.async_copy(
                w_hbm.at[idx_v.at[j]],
                rows_v.at[pl.ds(j * IDXW, IDXW)],
                sem,
            )
            for j in range(GATHERS)
        ]
        for cp in copies:
            cp.wait()

        def pe_body(l, inner_carry):
            pvs = [pe_v[l, pl.ds(j * 16, 16)] for j in range(NSUB)]
            for k in range(CHUNK // L):
                r = l + k * L
                for j in range(NSUB):
                    rows_v[r, pl.ds(j * 16, 16)] = (
                        rows_v[r, pl.ds(j * 16, 16)] * SCALE + pvs[j]
                    )
            return inner_carry

        lax.fori_loop(0, L, pe_body, 0)

        pltpu.sync_copy(rows_v, out_hbm.at[pl.ds(row0, CHUNK)])
        return carry

    lax.fori_loop(0, CHUNKS_PER_W, chunk_body, 0)


def kernel(x, W):
    pe = _pe_table()
    x_flat = x.reshape(ROWS // IDXW, IDXW)
    out = _sc_embed(x_flat, W, pe)
    return out.reshape(B, L, EMBED_DIM)


# speedup vs baseline: 1.1486x; 1.1486x over previous
import math
import functools

import jax
import jax.numpy as jnp
from jax import lax
from jax.experimental import pallas as pl
from jax.experimental.pallas import tpu as pltpu
from jax.experimental.pallas import tpu_sc as plsc

VOCAB = 1000000
EMBED_DIM = 64
WPAD = 128
B, L = 4096, 200
SCALE = math.sqrt(EMBED_DIM)

NC, NS = 2, 16
NW = NC * NS

ROWS = B * L
IDXW = 100
CHUNK = 400
GATHERS = CHUNK // IDXW
ROWS_PER_W = ROWS // NW
CHUNKS_PER_W = ROWS_PER_W // CHUNK
NSUB = EMBED_DIM // 16


def _pe_table():
    idx = jnp.arange(0, EMBED_DIM, 2, dtype=jnp.float32)
    pos = jnp.arange(0, L, dtype=jnp.float32)[:, None]
    div_term = jnp.exp(-idx / EMBED_DIM * math.log(10000.0))
    ang = pos * div_term
    pe = jnp.zeros((L, EMBED_DIM), dtype=jnp.float32)
    pe = pe.at[:, 0::2].set(jnp.sin(ang))
    pe = pe.at[:, 1::2].set(jnp.cos(ang))
    return pe


@functools.partial(
    pl.kernel,
    out_type=jax.ShapeDtypeStruct((ROWS, EMBED_DIM), jnp.float32),
    mesh=plsc.VectorSubcoreMesh(core_axis_name="c", subcore_axis_name="s"),
    scratch_types=[
        pltpu.VMEM((GATHERS, IDXW), jnp.int32),
        pltpu.VMEM((CHUNK, WPAD), jnp.float32),
        pltpu.VMEM((CHUNK, EMBED_DIM), jnp.float32),
        pltpu.VMEM((L, EMBED_DIM), jnp.float32),
        pltpu.SemaphoreType.DMA,
    ],
)
def _sc_embed(x_hbm, w_hbm, pe_hbm, out_hbm, idx_v, rows_v, out_v, pe_v, sem):
    wid = lax.axis_index("s") * NC + lax.axis_index("c")
    base_idxrow = wid * (ROWS_PER_W // IDXW)
    base_row = wid * ROWS_PER_W

    pltpu.sync_copy(pe_hbm, pe_v)

    def chunk_body(c, carry):
        idxrow0 = base_idxrow + c * GATHERS
        row0 = base_row + c * CHUNK

        pltpu.sync_copy(x_hbm.at[pl.ds(idxrow0, GATHERS)], idx_v)

        copies = [
            pltpu.async_copy(
                w_hbm.at[idx_v.at[j]],
                rows_v.at[pl.ds(j * IDXW, IDXW)],
                sem,
            )
            for j in range(GATHERS)
        ]
        for cp in copies:
            cp.wait()

        def pe_body(l, inner_carry):
            pvs = [pe_v[l, pl.ds(j * 16, 16)] for j in range(NSUB)]
            for k in range(CHUNK // L):
                r = l + k * L
                for j in range(NSUB):
                    out_v[r, pl.ds(j * 16, 16)] = (
                        rows_v[r, pl.ds(j * 16, 16)] * SCALE + pvs[j]
                    )
            return inner_carry

        lax.fori_loop(0, L, pe_body, 0)

        pltpu.sync_copy(out_v, out_hbm.at[pl.ds(row0, CHUNK)])
        return carry

    lax.fori_loop(0, CHUNKS_PER_W, chunk_body, 0)


def kernel(x, W):
    pe = _pe_table()
    w_pad = jnp.pad(W, ((0, 0), (0, WPAD - EMBED_DIM)))
    x_flat = x.reshape(ROWS // IDXW, IDXW)
    out = _sc_embed(x_flat, w_pad, pe)
    return out.reshape(B, L, EMBED_DIM)
